# Initial kernel scaffold; baseline (speedup 1.0000x reference)
import jax
import jax.numpy as jnp
from jax.experimental import pallas as pl


def kernel(x, a, edge_index, W1l, W1r, b1, W2l, W2r, b2, W3l, W3r, b3, Wf1, bf1, Wf2, bf2, Wf3, bf3):
    raise NotImplementedError("write your pallas kernel here")


# speedup vs baseline: 6.6256x; 6.6256x over previous
import functools

import jax
import jax.numpy as jnp
from jax.experimental import pallas as pl

_N = 49
_F3 = 12 * _N
_MLP = 256


def _fused_net(x_ref, a_ref, m1_ref, b1_ref, m2_ref, b2_ref, m3_ref, b3_ref,
               wf1_ref, wf1a_ref, bf1_ref, wf2_ref, bf2_ref, wf3_ref, bf3_ref,
               out_ref):
    f32 = jnp.float32
    h0 = x_ref[...]
    h1 = jnp.maximum(
        jnp.dot(h0, m1_ref[...], preferred_element_type=f32) + b1_ref[...], 0.0)
    h2 = jnp.maximum(
        jnp.dot(h1, m2_ref[...], preferred_element_type=f32) + b2_ref[...], 0.0)
    h3 = jnp.dot(h2, m3_ref[...], preferred_element_type=f32) + b3_ref[...]
    f1 = jnp.maximum(
        jnp.dot(h3, wf1_ref[...], preferred_element_type=f32)
        + a_ref[...] * wf1a_ref[...] + bf1_ref[...], 0.0)
    f2 = jnp.maximum(
        jnp.dot(f1, wf2_ref[...], preferred_element_type=f32) + bf2_ref[...], 0.0)
    out_ref[...] = (jnp.dot(f2, wf3_ref[...], preferred_element_type=f32)
                    + bf3_ref[...])


@functools.partial(jax.jit, static_argnames=())
def kernel(x, a, edge_index, W1l, W1r, b1, W2l, W2r, b2, W3l, W3r, b3,
           Wf1, bf1, Wf2, bf2, Wf3, bf3):
    B = x.shape[0]
    f32 = jnp.float32

    src, dst = edge_index[0], edge_index[1]
    adj = jnp.zeros((_N, _N), f32).at[dst, src].add(1.0)
    deg = adj.sum(axis=1)
    adj = adj / jnp.maximum(deg, 1.0)[:, None]
    eye = jnp.eye(_N, dtype=f32)

    def comb(Wl, Wr):
        return jnp.kron(adj.T, Wl.T) + jnp.kron(eye, Wr.T)

    m1 = comb(W1l, W1r)
    m2 = comb(W2l, W2r)
    m3 = comb(W3l, W3r)
    b1r = jnp.tile(b1, _N)[None, :]
    b2r = jnp.tile(b2, _N)[None, :]
    b3r = jnp.tile(b3, _N)[None, :]
    wf1 = Wf1[:, :_F3].T
    wf1a = Wf1[None, :, _F3]
    wf2 = Wf2.T
    wf3 = Wf3.T

    x2 = x.reshape(B, _N)

    bb = 2048
    grid = (B // bb,)

    def full(shape):
        return pl.BlockSpec(shape, lambda i: (0, 0))

    out = pl.pallas_call(
        _fused_net,
        grid=grid,
        in_specs=[
            pl.BlockSpec((bb, _N), lambda i: (i, 0)),
            pl.BlockSpec((bb, 1), lambda i: (i, 0)),
            full(m1.shape), full(b1r.shape),
            full(m2.shape), full(b2r.shape),
            full(m3.shape), full(b3r.shape),
            full(wf1.shape), full(wf1a.shape), full(bf1[None, :].shape),
            full(wf2.shape), full(bf2[None, :].shape),
            full(wf3.shape), full(bf3[None, :].shape),
        ],
        out_specs=pl.BlockSpec((bb, 1), lambda i: (i, 0)),
        out_shape=jax.ShapeDtypeStruct((B, 1), f32),
    )(x2, a, m1, b1r, m2, b2r, m3, b3r,
      wf1, wf1a, bf1[None, :], wf2, bf2[None, :], wf3, bf3[None, :])
    return out


# speedup vs baseline: 7.4313x; 1.1216x over previous
import functools

import jax
import jax.numpy as jnp
from jax.experimental import pallas as pl

_N = 49
_F3 = 12 * _N
_MLP = 256


def _fused_net(x_ref, a_ref, m1_ref, b1_ref, m2_ref, b2_ref, g_ref, wf1a_ref,
               c_ref, wf2_ref, bf2_ref, wf3_ref, bf3_ref, out_ref):
    f32 = jnp.float32
    h0 = x_ref[...]
    h1 = jnp.maximum(
        jnp.dot(h0, m1_ref[...], preferred_element_type=f32) + b1_ref[...], 0.0)
    h2 = jnp.maximum(
        jnp.dot(h1, m2_ref[...], preferred_element_type=f32) + b2_ref[...], 0.0)
    f1 = jnp.maximum(
        jnp.dot(h2, g_ref[...], preferred_element_type=f32)
        + a_ref[...] * wf1a_ref[...] + c_ref[...], 0.0)
    f2 = jnp.maximum(
        jnp.dot(f1, wf2_ref[...], preferred_element_type=f32) + bf2_ref[...], 0.0)
    out_ref[...] = (jnp.dot(f2, wf3_ref[...], preferred_element_type=f32)
                    + bf3_ref[...])


@functools.partial(jax.jit, static_argnames=())
def kernel(x, a, edge_index, W1l, W1r, b1, W2l, W2r, b2, W3l, W3r, b3,
           Wf1, bf1, Wf2, bf2, Wf3, bf3):
    B = x.shape[0]
    f32 = jnp.float32

    src, dst = edge_index[0], edge_index[1]
    adj = jnp.zeros((_N, _N), f32).at[dst, src].add(1.0)
    deg = adj.sum(axis=1)
    adj = adj / jnp.maximum(deg, 1.0)[:, None]
    eye = jnp.eye(_N, dtype=f32)

    def comb(Wl, Wr):
        return jnp.kron(adj.T, Wl.T) + jnp.kron(eye, Wr.T)

    m1 = comb(W1l, W1r)
    m2 = comb(W2l, W2r)
    m3 = comb(W3l, W3r)
    b1r = jnp.tile(b1, _N)[None, :]
    b2r = jnp.tile(b2, _N)[None, :]
    b3r = jnp.tile(b3, _N)[None, :]
    wf1 = Wf1[:, :_F3].T
    wf1a = Wf1[None, :, _F3]
    g = m3 @ wf1
    c = b3r @ wf1 + bf1[None, :]
    wf2 = Wf2.T
    wf3 = Wf3.T

    x2 = x.reshape(B, _N)

    bb = 2048
    grid = (B // bb,)

    def full(shape):
        return pl.BlockSpec(shape, lambda i: (0, 0))

    out = pl.pallas_call(
        _fused_net,
        grid=grid,
        in_specs=[
            pl.BlockSpec((bb, _N), lambda i: (i, 0)),
            pl.BlockSpec((bb, 1), lambda i: (i, 0)),
            full(m1.shape), full(b1r.shape),
            full(m2.shape), full(b2r.shape),
            full(g.shape), full(wf1a.shape), full(c.shape),
            full(wf2.shape), full(bf2[None, :].shape),
            full(wf3.shape), full(bf3[None, :].shape),
        ],
        out_specs=pl.BlockSpec((bb, 1), lambda i: (i, 0)),
        out_shape=jax.ShapeDtypeStruct((B, 1), f32),
    )(x2, a, m1, b1r, m2, b2r, g, wf1a, c,
      wf2, bf2[None, :], wf3, bf3[None, :])
    return out


# speedup vs baseline: 11.7096x; 1.5757x over previous
import functools

import jax
import jax.numpy as jnp
from jax.experimental import pallas as pl
from jax.experimental.pallas import tpu as pltpu

_N = 49
_E = 168
_F3 = 12 * _N
_MLP = 256


def _dot_t(x, y):
    return jax.lax.dot_general(x, y, (((1,), (1,)), ((), ())),
                               preferred_element_type=jnp.float32)


def _dot_tl(x, y):
    return jax.lax.dot_general(x, y, (((0,), (0,)), ((), ())),
                               preferred_element_type=jnp.float32)


def _iota2(shape, dim):
    return jax.lax.broadcasted_iota(jnp.int32, shape, dim)


def _fused_net(x_ref, a_ref, ei_ref, w1l_ref, w1r_ref, b1_ref, w2l_ref,
               w2r_ref, b2_ref, w3l_ref, w3r_ref, b3_ref, wf1_ref, bf1_ref,
               wf2_ref, bf2_ref, wf3_ref, bf3_ref, out_ref,
               m1_s, b1_s, m2_s, b2_s, g_s, wa_s, c_s):
    f32 = jnp.float32

    @pl.when(pl.program_id(0) == 0)
    def _prep():
        src = ei_ref[0:1, :]
        dst = ei_ref[1:2, :]
        dmat = (_iota2((_N, _E), 0) == dst).astype(f32)
        smat = (_iota2((_N, _E), 0) == src).astype(f32)
        adj = _dot_t(dmat, smat)
        deg = jnp.sum(adj, axis=1, keepdims=True)
        adj = adj / jnp.maximum(deg, 1.0)

        pt6 = (_iota2((_N, 6 * _N), 1) // 6 == _iota2((_N, 6 * _N), 0)).astype(f32)
        qt6 = (_iota2((6, 6 * _N), 1) % 6 == _iota2((6, 6 * _N), 0)).astype(f32)
        pt12 = (_iota2((_N, _F3), 1) // 12 == _iota2((_N, _F3), 0)).astype(f32)
        qt12 = (_iota2((12, _F3), 1) % 12 == _iota2((12, _F3), 0)).astype(f32)

        ka1 = _dot_tl(adj, pt6)
        w1l_row = _dot_tl(w1l_ref[...], qt6)
        w1r_row = _dot_tl(w1r_ref[...], qt6)
        m1_s[...] = ka1 * w1l_row + pt6 * w1r_row
        b1_s[...] = jnp.dot(b1_ref[...], qt6, preferred_element_type=f32)

        ka2 = _dot_tl(pt6, _dot_tl(adj, pt6))
        w2l_e = _dot_tl(qt6, _dot_tl(w2l_ref[...], qt6))
        w2r_e = _dot_tl(qt6, _dot_tl(w2r_ref[...], qt6))
        bm6 = (_iota2((6 * _N, 6 * _N), 0) // 6
               == _iota2((6 * _N, 6 * _N), 1) // 6).astype(f32)
        m2_s[...] = ka2 * w2l_e + bm6 * w2r_e
        b2_s[...] = jnp.dot(b2_ref[...], qt6, preferred_element_type=f32)

        ka3 = _dot_tl(pt6, _dot_tl(adj, pt12))
        w3l_e = _dot_tl(qt6, _dot_tl(w3l_ref[...], qt12))
        w3r_e = _dot_tl(qt6, _dot_tl(w3r_ref[...], qt12))
        bm612 = (_iota2((6 * _N, _F3), 0) // 6
                 == _iota2((6 * _N, _F3), 1) // 12).astype(f32)
        m3 = ka3 * w3l_e + bm612 * w3r_e
        wf1m = wf1_ref[:, :_F3]
        g_s[...] = _dot_t(m3, wf1m)
        b3_row = jnp.dot(b3_ref[...], qt12, preferred_element_type=f32)
        c_s[...] = _dot_t(b3_row, wf1m) + bf1_ref[...]
        wa_s[...] = jax.lax.dot_general(
            jnp.ones((1, 1), f32), wf1_ref[:, _F3:],
            (((0,), (1,)), ((), ())), preferred_element_type=f32)

    h0 = x_ref[...]
    h1 = jnp.maximum(
        jnp.dot(h0, m1_s[...], preferred_element_type=f32) + b1_s[...], 0.0)
    h2 = jnp.maximum(
        jnp.dot(h1, m2_s[...], preferred_element_type=f32) + b2_s[...], 0.0)
    f1 = jnp.maximum(
        jnp.dot(h2, g_s[...], preferred_element_type=f32)
        + a_ref[...] * wa_s[...] + c_s[...], 0.0)
    f2 = jnp.maximum(
        _dot_t(f1, wf2_ref[...]) + bf2_ref[...], 0.0)
    out_ref[...] = (jnp.sum(f2 * wf3_ref[...], axis=1, keepdims=True)
                    + bf3_ref[...])


@functools.partial(jax.jit, static_argnames=())
def kernel(x, a, edge_index, W1l, W1r, b1, W2l, W2r, b2, W3l, W3r, b3,
           Wf1, bf1, Wf2, bf2, Wf3, bf3):
    B = x.shape[0]
    f32 = jnp.float32
    x2 = x.reshape(B, _N)

    bb = 2048
    grid = (B // bb,)

    def full(arr):
        return pl.BlockSpec(arr.shape, lambda i: tuple(0 for _ in arr.shape))

    args = (x2, a, edge_index, W1l, W1r, b1[None, :], W2l, W2r, b2[None, :],
            W3l, W3r, b3[None, :], Wf1, bf1[None, :], Wf2, bf2[None, :],
            Wf3, bf3[None, :])
    in_specs = [
        pl.BlockSpec((bb, _N), lambda i: (i, 0)),
        pl.BlockSpec((bb, 1), lambda i: (i, 0)),
    ] + [full(t) for t in args[2:]]

    out = pl.pallas_call(
        _fused_net,
        grid=grid,
        in_specs=in_specs,
        out_specs=pl.BlockSpec((bb, 1), lambda i: (i, 0)),
        out_shape=jax.ShapeDtypeStruct((B, 1), f32),
        scratch_shapes=[
            pltpu.VMEM((_N, 6 * _N), f32),
            pltpu.VMEM((1, 6 * _N), f32),
            pltpu.VMEM((6 * _N, 6 * _N), f32),
            pltpu.VMEM((1, 6 * _N), f32),
            pltpu.VMEM((6 * _N, _MLP), f32),
            pltpu.VMEM((1, _MLP), f32),
            pltpu.VMEM((1, _MLP), f32),
        ],
    )(*args)
    return out


# speedup vs baseline: 12.0309x; 1.0274x over previous
import functools

import jax
import jax.numpy as jnp
from jax.experimental import pallas as pl
from jax.experimental.pallas import tpu as pltpu

_N = 49
_E = 168
_F3 = 12 * _N
_MLP = 256


def _dot_t(x, y):
    return jax.lax.dot_general(x, y, (((1,), (1,)), ((), ())),
                               preferred_element_type=jnp.float32)


def _dot_tl(x, y):
    return jax.lax.dot_general(x, y, (((0,), (0,)), ((), ())),
                               preferred_element_type=jnp.float32)


def _iota2(shape, dim):
    return jax.lax.broadcasted_iota(jnp.int32, shape, dim)


def _fused_net(x_ref, a_ref, ei_ref, w1l_ref, w1r_ref, b1_ref, w2l_ref,
               w2r_ref, b2_ref, w3l_ref, w3r_ref, b3_ref, wf1_ref, bf1_ref,
               wf2_ref, bf2_ref, wf3_ref, bf3_ref, out_ref,
               m1_s, b1_s, m2_s, b2_s, g_s, wa_s, c_s):
    f32 = jnp.float32

    @pl.when(pl.program_id(0) == 0)
    def _prep():
        src = ei_ref[0:1, :]
        dst = ei_ref[1:2, :]
        dmat = (_iota2((_N, _E), 0) == dst).astype(f32)
        smat = (_iota2((_N, _E), 0) == src).astype(f32)
        adj = _dot_t(dmat, smat)
        deg = jnp.sum(adj, axis=1, keepdims=True)
        adj = adj / jnp.maximum(deg, 1.0)

        pt6 = (_iota2((_N, 6 * _N), 1) // 6 == _iota2((_N, 6 * _N), 0)).astype(f32)
        qt6 = (_iota2((6, 6 * _N), 1) % 6 == _iota2((6, 6 * _N), 0)).astype(f32)
        pt12 = (_iota2((_N, _F3), 1) // 12 == _iota2((_N, _F3), 0)).astype(f32)
        qt12 = (_iota2((12, _F3), 1) % 12 == _iota2((12, _F3), 0)).astype(f32)

        ka1 = _dot_tl(adj, pt6)
        w1l_row = _dot_tl(w1l_ref[...], qt6)
        w1r_row = _dot_tl(w1r_ref[...], qt6)
        m1_s[...] = ka1 * w1l_row + pt6 * w1r_row
        b1_s[...] = jnp.dot(b1_ref[...], qt6, preferred_element_type=f32)

        ka2 = _dot_tl(pt6, _dot_tl(adj, pt6))
        w2l_e = _dot_tl(qt6, _dot_tl(w2l_ref[...], qt6))
        w2r_e = _dot_tl(qt6, _dot_tl(w2r_ref[...], qt6))
        bm6 = (_iota2((6 * _N, 6 * _N), 0) // 6
               == _iota2((6 * _N, 6 * _N), 1) // 6).astype(f32)
        m2_s[...] = ka2 * w2l_e + bm6 * w2r_e
        b2_s[...] = jnp.dot(b2_ref[...], qt6, preferred_element_type=f32)

        ka3 = _dot_tl(pt6, _dot_tl(adj, pt12))
        w3l_e = _dot_tl(qt6, _dot_tl(w3l_ref[...], qt12))
        w3r_e = _dot_tl(qt6, _dot_tl(w3r_ref[...], qt12))
        bm612 = (_iota2((6 * _N, _F3), 0) // 6
                 == _iota2((6 * _N, _F3), 1) // 12).astype(f32)
        m3 = ka3 * w3l_e + bm612 * w3r_e
        wf1m = wf1_ref[:, :_F3]
        g_s[...] = _dot_t(m3, wf1m)
        b3_row = jnp.dot(b3_ref[...], qt12, preferred_element_type=f32)
        c_s[...] = _dot_t(b3_row, wf1m) + bf1_ref[...]
        wa_s[...] = jax.lax.dot_general(
            jnp.ones((1, 1), f32), wf1_ref[:, _F3:],
            (((0,), (1,)), ((), ())), preferred_element_type=f32)

    h0 = x_ref[...]
    h1 = jnp.maximum(
        jnp.dot(h0, m1_s[...], preferred_element_type=f32) + b1_s[...], 0.0)
    h2 = jnp.maximum(
        jnp.dot(h1, m2_s[...], preferred_element_type=f32) + b2_s[...], 0.0)
    f1 = jnp.maximum(
        jnp.dot(h2, g_s[...], preferred_element_type=f32)
        + a_ref[...] * wa_s[...] + c_s[...], 0.0)
    f2 = jnp.maximum(
        _dot_t(f1, wf2_ref[...]) + bf2_ref[...], 0.0)
    out_ref[...] = (jnp.sum(f2 * wf3_ref[...], axis=1, keepdims=True)
                    + bf3_ref[...])


@functools.partial(jax.jit, static_argnames=())
def kernel(x, a, edge_index, W1l, W1r, b1, W2l, W2r, b2, W3l, W3r, b3,
           Wf1, bf1, Wf2, bf2, Wf3, bf3):
    B = x.shape[0]
    f32 = jnp.float32
    x2 = x.reshape(B, _N)

    bb = 4096
    grid = (B // bb,)

    def full(arr):
        return pl.BlockSpec(arr.shape, lambda i: tuple(0 for _ in arr.shape))

    args = (x2, a, edge_index, W1l, W1r, b1[None, :], W2l, W2r, b2[None, :],
            W3l, W3r, b3[None, :], Wf1, bf1[None, :], Wf2, bf2[None, :],
            Wf3, bf3[None, :])
    in_specs = [
        pl.BlockSpec((bb, _N), lambda i: (i, 0)),
        pl.BlockSpec((bb, 1), lambda i: (i, 0)),
    ] + [full(t) for t in args[2:]]

    out = pl.pallas_call(
        _fused_net,
        grid=grid,
        in_specs=in_specs,
        out_specs=pl.BlockSpec((bb, 1), lambda i: (i, 0)),
        out_shape=jax.ShapeDtypeStruct((B, 1), f32),
        scratch_shapes=[
            pltpu.VMEM((_N, 6 * _N), f32),
            pltpu.VMEM((1, 6 * _N), f32),
            pltpu.VMEM((6 * _N, 6 * _N), f32),
            pltpu.VMEM((1, 6 * _N), f32),
            pltpu.VMEM((6 * _N, _MLP), f32),
            pltpu.VMEM((1, _MLP), f32),
            pltpu.VMEM((1, _MLP), f32),
        ],
    )(*args)
    return out
